# baseline (speedup 1.0000x reference)
import jax
import jax.numpy as jnp
from jax.experimental import pallas as pl
from jax.experimental.pallas import tpu as pltpu

_NS = 5


def _gcn_kernel(*refs):
    a_refs = refs[:_NS]
    f_ref, n_ref, w_ref, b2_ref, alpha_ref = refs[_NS:_NS + 5]
    h_refs = refs[_NS + 5:2 * _NS + 5]
    s_ref = refs[2 * _NS + 5]
    F = w_ref.shape[1]

    @pl.when(pl.program_id(0) == 0)
    def _():
        w = w_ref[:]
        s_ref[:, :F] = jnp.dot(
            f_ref[:], w, preferred_element_type=jnp.float32
        ).astype(jnp.bfloat16)
        s_ref[:, F:] = jnp.dot(
            n_ref[:], w, preferred_element_type=jnp.float32
        ).astype(jnp.bfloat16)

    s = s_ref[:]
    b2 = b2_ref[:]
    alpha = alpha_ref[0, 0]

    for a_ref, h_ref in zip(a_refs, h_refs):
        acc = jnp.dot(a_ref[0].astype(jnp.bfloat16), s,
                      preferred_element_type=jnp.float32)
        h = acc + b2
        h_ref[0] = jnp.where(h >= 0, h, alpha * h).astype(jnp.bfloat16)


def _disc_kernel(*refs):
    h_refs = refs[:_NS]
    dwt_ref, db_ref = refs[_NS], refs[_NS + 1]
    sc1_refs = refs[_NS + 2:2 * _NS + 2]
    sc2_refs = refs[2 * _NS + 2:]
    F = dwt_ref.shape[0]
    n = sum(h.shape[0] for h in h_refs)
    hps = [h[:, :F].astype(jnp.float32) for h in h_refs]
    hns = [h[:, F:].astype(jnp.float32) for h in h_refs]
    c = sum(jnp.sum(hp, axis=0, keepdims=True) for hp in hps) * (1.0 / n)
    sg = jax.nn.sigmoid(c)
    u = jnp.dot(sg, dwt_ref[:], preferred_element_type=jnp.float32)
    db = db_ref[0, 0]
    for hp, hn, sc1_ref, sc2_ref in zip(hps, hns, sc1_refs, sc2_refs):
        sc1_ref[:] = jnp.sum(hp * u, axis=1, keepdims=True) + db
        sc2_ref[:] = jnp.sum(hn * u, axis=1, keepdims=True) + db


def kernel(features, negative_features, adj, W_gcn, b_gcn, prelu_alpha, disc_W, disc_b):
    B, N, IN_F = features.shape
    OUT_F = W_gcn.shape[1]
    G = N // _NS
    f2 = features.reshape(N, IN_F)
    n2 = negative_features.reshape(N, IN_F)
    adj3 = adj.reshape(_NS, G, N)
    b2 = jnp.concatenate([b_gcn, b_gcn]).reshape(1, 2 * OUT_F)
    alpha = prelu_alpha.reshape(1, 1)
    db = disc_b.reshape(1, 1)
    dwt = disc_W.T

    BM = 80
    aspec = lambda g: pl.BlockSpec((1, BM, N), lambda i, g=g: (g, i, 0))
    hspec = pl.BlockSpec((1, BM, 2 * OUT_F), lambda i: (0, i, 0))
    hs = pl.pallas_call(
        _gcn_kernel,
        grid=(G // BM,),
        in_specs=[aspec(g) for g in range(_NS)] + [
            pl.BlockSpec((N, IN_F), lambda i: (0, 0)),
            pl.BlockSpec((N, IN_F), lambda i: (0, 0)),
            pl.BlockSpec((IN_F, OUT_F), lambda i: (0, 0)),
            pl.BlockSpec((1, 2 * OUT_F), lambda i: (0, 0)),
            pl.BlockSpec((1, 1), lambda i: (0, 0)),
        ],
        out_specs=[hspec] * _NS,
        out_shape=[jax.ShapeDtypeStruct((1, G, 2 * OUT_F), jnp.bfloat16)] * _NS,
        scratch_shapes=[pltpu.VMEM((N, 2 * OUT_F), jnp.bfloat16)],
    )(*([adj3] * _NS), f2, n2, W_gcn, b2, alpha)

    scs = pl.pallas_call(
        _disc_kernel,
        out_shape=[jax.ShapeDtypeStruct((G, 1), jnp.float32)] * (2 * _NS),
    )(*[h.reshape(G, 2 * OUT_F) for h in hs], dwt, db)

    return jnp.concatenate([s.reshape(1, G) for s in scs], axis=1)


# speedup vs baseline: 1.0209x; 1.0209x over previous
import jax
import jax.numpy as jnp
from jax.experimental import pallas as pl
from jax.experimental.pallas import tpu as pltpu


def _seq_kernel(f_ref, n_ref, w_ref, s_ref):
    w = w_ref[:]
    F = w.shape[1]
    s_ref[:, :F] = jnp.dot(
        f_ref[:], w, preferred_element_type=jnp.float32
    ).astype(jnp.bfloat16)
    s_ref[:, F:] = jnp.dot(
        n_ref[:], w, preferred_element_type=jnp.float32
    ).astype(jnp.bfloat16)


def _gcn_kernel(adj_t_ref, adj_b_ref, s_ref, b2_ref, alpha_ref, ht_ref, hb_ref):
    s = s_ref[:]
    b2 = b2_ref[:]
    alpha = alpha_ref[0, 0]

    def mm(a_ref, o_ref):
        acc = jnp.dot(a_ref[0].astype(jnp.bfloat16), s,
                      preferred_element_type=jnp.float32)
        h = acc + b2
        o_ref[0] = jnp.where(h >= 0, h, alpha * h).astype(jnp.bfloat16)

    mm(adj_t_ref, ht_ref)
    mm(adj_b_ref, hb_ref)


def _disc_kernel(ht_ref, hb_ref, dwt_ref, db_ref,
                 sc1t_ref, sc1b_ref, sc2t_ref, sc2b_ref):
    n = ht_ref.shape[0] + hb_ref.shape[0]
    F = dwt_ref.shape[0]
    hpt = ht_ref[:, :F].astype(jnp.float32)
    hnt = ht_ref[:, F:].astype(jnp.float32)
    hpb = hb_ref[:, :F].astype(jnp.float32)
    hnb = hb_ref[:, F:].astype(jnp.float32)
    c = (jnp.sum(hpt, axis=0, keepdims=True)
         + jnp.sum(hpb, axis=0, keepdims=True)) * (1.0 / n)
    sg = jax.nn.sigmoid(c)
    u = jnp.dot(sg, dwt_ref[:], preferred_element_type=jnp.float32)
    db = db_ref[0, 0]
    sc1t_ref[:] = jnp.sum(hpt * u, axis=1, keepdims=True) + db
    sc1b_ref[:] = jnp.sum(hpb * u, axis=1, keepdims=True) + db
    sc2t_ref[:] = jnp.sum(hnt * u, axis=1, keepdims=True) + db
    sc2b_ref[:] = jnp.sum(hnb * u, axis=1, keepdims=True) + db


def kernel(features, negative_features, adj, W_gcn, b_gcn, prelu_alpha, disc_W, disc_b):
    B, N, IN_F = features.shape
    OUT_F = W_gcn.shape[1]
    H = N // 2
    f2 = features.reshape(N, IN_F)
    n2 = negative_features.reshape(N, IN_F)
    adj3 = adj.reshape(2, H, N)
    b2 = jnp.concatenate([b_gcn, b_gcn]).reshape(1, 2 * OUT_F)
    alpha = prelu_alpha.reshape(1, 1)
    db = disc_b.reshape(1, 1)
    dwt = disc_W.T

    S = pl.pallas_call(
        _seq_kernel,
        out_shape=jax.ShapeDtypeStruct((N, 2 * OUT_F), jnp.bfloat16),
    )(f2, n2, W_gcn)

    BM = 200
    ht, hb = pl.pallas_call(
        _gcn_kernel,
        grid=(H // BM,),
        in_specs=[
            pl.BlockSpec((1, BM, N), lambda i: (0, i, 0)),
            pl.BlockSpec((1, BM, N), lambda i: (1, i, 0)),
            pl.BlockSpec((N, 2 * OUT_F), lambda i: (0, 0)),
            pl.BlockSpec((1, 2 * OUT_F), lambda i: (0, 0)),
            pl.BlockSpec((1, 1), lambda i: (0, 0)),
        ],
        out_specs=[
            pl.BlockSpec((1, BM, 2 * OUT_F), lambda i: (0, i, 0)),
            pl.BlockSpec((1, BM, 2 * OUT_F), lambda i: (0, i, 0)),
        ],
        out_shape=[
            jax.ShapeDtypeStruct((1, H, 2 * OUT_F), jnp.bfloat16),
            jax.ShapeDtypeStruct((1, H, 2 * OUT_F), jnp.bfloat16),
        ],
        compiler_params=pltpu.CompilerParams(
            dimension_semantics=("parallel",)),
    )(adj3, adj3, S, b2, alpha)

    sc1t, sc1b, sc2t, sc2b = pl.pallas_call(
        _disc_kernel,
        out_shape=[
            jax.ShapeDtypeStruct((H, 1), jnp.float32),
            jax.ShapeDtypeStruct((H, 1), jnp.float32),
            jax.ShapeDtypeStruct((H, 1), jnp.float32),
            jax.ShapeDtypeStruct((H, 1), jnp.float32),
        ],
    )(ht.reshape(H, 2 * OUT_F), hb.reshape(H, 2 * OUT_F), dwt, db)

    return jnp.concatenate(
        [sc1t.reshape(1, H), sc1b.reshape(1, H),
         sc2t.reshape(1, H), sc2b.reshape(1, H)], axis=1)


# speedup vs baseline: 1.0808x; 1.0586x over previous
import jax
import jax.numpy as jnp
from jax.experimental import pallas as pl
from jax.experimental.pallas import tpu as pltpu


def _dgi_kernel(adj_t_ref, adj_b_ref, f_ref, n_ref, w_ref, b2_ref, alpha_ref,
                dwt_ref, db_ref,
                sc1t_ref, sc1b_ref, sc2t_ref, sc2b_ref,
                s_ref, ht_ref, hb_ref, csum_ref):
    F = w_ref.shape[1]
    i = pl.program_id(0)
    nsteps = pl.num_programs(0)
    BM = adj_t_ref.shape[1]

    @pl.when(i == 0)
    def _():
        w = w_ref[:]
        s_ref[:, :F] = jnp.dot(
            f_ref[:], w, preferred_element_type=jnp.float32
        ).astype(jnp.bfloat16)
        s_ref[:, F:] = jnp.dot(
            n_ref[:], w, preferred_element_type=jnp.float32
        ).astype(jnp.bfloat16)
        csum_ref[:] = jnp.zeros_like(csum_ref)

    s = s_ref[:]
    b2 = b2_ref[:]
    alpha = alpha_ref[0, 0]
    row = i * BM

    def mm(a_ref, h_all_ref):
        acc = jnp.dot(a_ref[0].astype(jnp.bfloat16), s,
                      preferred_element_type=jnp.float32)
        hblk = acc + b2
        hblk = jnp.where(hblk >= 0, hblk, alpha * hblk)
        h_all_ref[pl.ds(row, BM), :] = hblk.astype(jnp.bfloat16)
        return jnp.sum(hblk[:, :F], axis=0, keepdims=True)

    cs_t = mm(adj_t_ref, ht_ref)
    cs_b = mm(adj_b_ref, hb_ref)
    csum_ref[:] = csum_ref[:] + cs_t + cs_b

    @pl.when(i == nsteps - 1)
    def _():
        n_nodes = 2 * ht_ref.shape[0]
        c = csum_ref[:] * (1.0 / n_nodes)
        sg = jax.nn.sigmoid(c)
        u = jnp.dot(sg, dwt_ref[:], preferred_element_type=jnp.float32)
        db = db_ref[0, 0]

        def scores(h_all_ref, sc1_out, sc2_out):
            hp = h_all_ref[:, :F].astype(jnp.float32)
            hn = h_all_ref[:, F:].astype(jnp.float32)
            sc1_out[:] = jnp.sum(hp * u, axis=1, keepdims=True) + db
            sc2_out[:] = jnp.sum(hn * u, axis=1, keepdims=True) + db

        scores(ht_ref, sc1t_ref, sc2t_ref)
        scores(hb_ref, sc1b_ref, sc2b_ref)


def kernel(features, negative_features, adj, W_gcn, b_gcn, prelu_alpha, disc_W, disc_b):
    B, N, IN_F = features.shape
    OUT_F = W_gcn.shape[1]
    H = N // 2
    f2 = features.reshape(N, IN_F)
    n2 = negative_features.reshape(N, IN_F)
    adj3 = adj.reshape(2, H, N)
    b2 = jnp.concatenate([b_gcn, b_gcn]).reshape(1, 2 * OUT_F)
    alpha = prelu_alpha.reshape(1, 1)
    db = disc_b.reshape(1, 1)
    dwt = disc_W.T

    BM = 200
    const = lambda shape: pl.BlockSpec(shape, lambda i: tuple(0 for _ in shape))
    sc1t, sc1b, sc2t, sc2b = pl.pallas_call(
        _dgi_kernel,
        grid=(H // BM,),
        in_specs=[
            pl.BlockSpec((1, BM, N), lambda i: (0, i, 0)),
            pl.BlockSpec((1, BM, N), lambda i: (1, i, 0)),
            const((N, IN_F)),
            const((N, IN_F)),
            const((IN_F, OUT_F)),
            const((1, 2 * OUT_F)),
            const((1, 1)),
            const((OUT_F, OUT_F)),
            const((1, 1)),
        ],
        out_specs=[const((H, 1))] * 4,
        out_shape=[jax.ShapeDtypeStruct((H, 1), jnp.float32)] * 4,
        scratch_shapes=[
            pltpu.VMEM((N, 2 * OUT_F), jnp.bfloat16),
            pltpu.VMEM((H, 2 * OUT_F), jnp.bfloat16),
            pltpu.VMEM((H, 2 * OUT_F), jnp.bfloat16),
            pltpu.VMEM((1, OUT_F), jnp.float32),
        ],
        compiler_params=pltpu.CompilerParams(
            vmem_limit_bytes=100 * 1024 * 1024),
    )(adj3, adj3, f2, n2, W_gcn, b2, alpha, dwt, db)

    return jnp.concatenate(
        [sc1t.reshape(1, H), sc1b.reshape(1, H),
         sc2t.reshape(1, H), sc2b.reshape(1, H)], axis=1)
